# Initial kernel scaffold; baseline (speedup 1.0000x reference)
import jax
import jax.numpy as jnp
from jax.experimental import pallas as pl


def kernel(queries, keys, k):
    raise NotImplementedError("write your pallas kernel here")


# speedup vs baseline: 2.9759x; 2.9759x over previous
import functools

import jax
import jax.numpy as jnp
from jax import lax
from jax.experimental import pallas as pl
from jax.experimental.pallas import tpu as pltpu
from jax.experimental.pallas import tpu_sc as plsc

Q = 1024
K = 100000
D = 64
TOPK = 10

QB = 256
KB = 2048
NKB = (K + KB - 1) // KB
GROUP = 16
GPB = KB // GROUP
NG = NKB * GPB
NCAND = TOPK * GROUP
AUGW = GROUP * D + 128
QCB = 64
NEG = float("-inf")
BIGI = 1 << 30


def _ab_body(q_ref, k_ref, kn_ref, qn_ref, outT_ref, ct_ref):
    kb = pl.program_id(1)
    kblk = k_ref[...]
    qblk = q_ref[...]
    simsT = lax.dot_general(kblk.astype(jnp.bfloat16),
                            qblk.astype(jnp.bfloat16),
                            (((1,), (1,)), ((), ())),
                            preferred_element_type=jnp.float32)
    denom = kn_ref[...] * qn_ref[...] + 1e-6
    normed = simsT * (1.0 / denom)
    rid = lax.broadcasted_iota(jnp.int32, (KB, QB), 0) + kb * KB
    x = jnp.where(rid < K, normed, NEG)
    x3 = x.reshape(GPB, GROUP, QB)
    acc = x3[:, 0:1, :]
    for j in range(1, GROUP):
        acc = jnp.maximum(acc, x3[:, j:j + 1, :])
    ct_ref[pl.ds(kb * GPB, GPB), :] = acc.reshape(GPB, QB)

    @pl.when(kb == NKB - 1)
    def _():
        cur = ct_ref[...]
        gidio = lax.broadcasted_iota(jnp.int32, (NG, QB), 0)
        sels = []
        for _t in range(TOPK):
            m = jnp.max(cur, axis=0, keepdims=True)
            sel = jnp.min(jnp.where(cur == m, gidio, BIGI),
                          axis=0, keepdims=True)
            sels.append(sel)
            cur = jnp.where(gidio == sel, NEG, cur)
        outT_ref[...] = jnp.concatenate(sels, axis=0)


_stage_ab = pl.pallas_call(
    _ab_body,
    grid=(Q // QB, NKB),
    in_specs=[
        pl.BlockSpec((QB, D), lambda qb, kb: (qb, 0)),
        pl.BlockSpec((KB, D), lambda qb, kb: (kb, 0)),
        pl.BlockSpec((KB, 1), lambda qb, kb: (kb, 0)),
        pl.BlockSpec((1, QB), lambda qb, kb: (0, qb)),
    ],
    out_specs=pl.BlockSpec((TOPK, QB), lambda qb, kb: (0, qb)),
    out_shape=jax.ShapeDtypeStruct((TOPK, Q), jnp.int32),
    scratch_shapes=[pltpu.VMEM((NG, QB), jnp.float32)],
    compiler_params=pltpu.CompilerParams(
        dimension_semantics=("arbitrary", "arbitrary")),
)


def _c_body(q_ref, gk_ref, kn_ref, gid_ref, qn_ref, vals_ref, idx_ref):
    q = q_ref[...]
    gid = gid_ref[...]
    keys3 = gk_ref[...].reshape(QCB * NCAND, D)
    s_all = lax.dot_general(keys3.astype(jnp.bfloat16),
                            q.astype(jnp.bfloat16),
                            (((1,), (1,)), ((), ())),
                            preferred_element_type=jnp.float32)
    s4 = s_all.reshape(QCB, TOPK, GROUP, QCB)
    bi = lax.broadcasted_iota(jnp.int32, (QCB, TOPK, GROUP, QCB), 0)
    qi = lax.broadcasted_iota(jnp.int32, (QCB, TOPK, GROUP, QCB), 3)
    dots = jnp.max(jnp.where(bi == qi, s4, NEG), axis=3)
    kn = kn_ref[...]
    qn = qn_ref[...][:, :, None]
    denom = qn * kn + 1e-6
    v = dots * (1.0 / denom)
    jio = lax.broadcasted_iota(jnp.int32, (QCB, TOPK, GROUP), 2)
    ci = gid[:, :, None] * GROUP + jio
    for t in range(TOPK):
        m2 = jnp.max(jnp.max(v, axis=2), axis=1, keepdims=True)
        m = m2[:, :, None]
        cand = jnp.where(v == m, ci, BIGI)
        sel2 = jnp.min(jnp.min(cand, axis=2), axis=1, keepdims=True)
        sel = sel2[:, :, None]
        v = jnp.where(ci == sel, NEG, v)
        vals_ref[:, t:t + 1] = jnp.where(m2 != 0.0, m2, 0.0)
        idx_ref[:, t:t + 1] = sel2


_stage_c = pl.pallas_call(
    _c_body,
    grid=(Q // QCB,),
    in_specs=[
        pl.BlockSpec((QCB, D), lambda i: (i, 0)),
        pl.BlockSpec((QCB, NCAND, D), lambda i: (i, 0, 0)),
        pl.BlockSpec((QCB, TOPK, GROUP), lambda i: (i, 0, 0)),
        pl.BlockSpec((QCB, TOPK), lambda i: (i, 0)),
        pl.BlockSpec((QCB, 1), lambda i: (i, 0)),
    ],
    out_specs=[
        pl.BlockSpec((QCB, TOPK), lambda i: (i, 0)),
        pl.BlockSpec((QCB, TOPK), lambda i: (i, 0)),
    ],
    out_shape=[
        jax.ShapeDtypeStruct((Q, TOPK), jnp.float32),
        jax.ShapeDtypeStruct((Q, TOPK), jnp.int32),
    ],
)


_NC, _NS = 2, 16
_NW = _NC * _NS
_B = Q * TOPK
_BPW = _B // _NW
_CH = 64
_NCH = _BPW // _CH


def _sc_gather_body(aug_hbm, idx_hbm, out_hbm, idx_v, rows_v, sem):
    wid = lax.axis_index("s") * _NC + lax.axis_index("c")

    def body(c, _):
        base = wid * _BPW + c * _CH
        pltpu.sync_copy(idx_hbm.at[pl.ds(base, _CH)], idx_v)
        pltpu.async_copy(aug_hbm.at[idx_v], rows_v, sem).wait()
        pltpu.sync_copy(rows_v, out_hbm.at[pl.ds(base, _CH)])
        return 0

    lax.fori_loop(0, _NCH, body, 0)


@functools.cache
def _sc_gather():
    return functools.partial(
        pl.kernel,
        mesh=plsc.VectorSubcoreMesh(core_axis_name="c", subcore_axis_name="s"),
        out_type=jax.ShapeDtypeStruct((_B, AUGW), jnp.float32),
        scratch_types=[
            pltpu.VMEM((_CH,), jnp.int32),
            pltpu.VMEM((_CH, AUGW), jnp.float32),
            pltpu.SemaphoreType.DMA,
        ],
    )(_sc_gather_body)


def kernel(queries, keys, k):
    kn = jnp.sqrt(jnp.sum(keys * keys, axis=1))
    qn = jnp.sqrt(jnp.sum(queries * queries, axis=1))
    kn_col = jnp.pad(kn, (0, NKB * KB - K)).reshape(NKB * KB, 1)
    qn_row = qn.reshape(1, Q)

    gidT = _stage_ab(queries, keys, kn_col, qn_row)
    gid = gidT.T

    aug = jnp.concatenate(
        [keys.reshape(K // GROUP, GROUP * D),
         jnp.pad(kn.reshape(K // GROUP, GROUP), ((0, 0), (0, 112)))], axis=1)
    gathered = _sc_gather()(aug, gid.reshape(-1))
    gk = gathered[:, :GROUP * D].reshape(Q, NCAND, D)
    kn_g = gathered[:, GROUP * D:GROUP * D + GROUP].reshape(Q, TOPK, GROUP)

    vals, idx = _stage_c(queries, gk, kn_g, gid, qn.reshape(Q, 1))
    vals = jnp.where(jnp.arange(TOPK)[None, :] < k, vals, 0.0)
    return vals, idx
